# baseline (speedup 1.0000x reference)
import functools

import jax
import jax.numpy as jnp
from jax.experimental import pallas as pl
from jax.experimental.pallas import tpu as pltpu


def _ln(h, g, b):
    f = h.shape[-1]
    ones_col = jnp.full((f, 1), 1.0 / f, jnp.float32)
    m = jnp.dot(h, ones_col, preferred_element_type=jnp.float32)
    msq = jnp.dot(h * h, ones_col, preferred_element_type=jnp.float32)
    v = msq - m * m
    return (h - m) * jax.lax.rsqrt(v + 1e-5) * g + b


def _shift_cat(x):
    z = jnp.zeros((1, x.shape[1]), x.dtype)
    prv = jnp.concatenate([z, x[:-1]], axis=0)
    nxt = jnp.concatenate([x[1:], z], axis=0)
    return jnp.concatenate([prv, x, nxt], axis=1)


def _vp_block(x, w1c, b1, g1, be1, w2c, b2, g2, be2, lw):
    h = jnp.dot(_shift_cat(x), w1c, preferred_element_type=jnp.float32) + b1
    h = _ln(jnp.maximum(h, 0.0), g1, be1)
    h = jnp.dot(_shift_cat(h), w2c, preferred_element_type=jnp.float32) + b2
    h = _ln(jnp.maximum(h, 0.0), g2, be2)
    return jnp.dot(h, lw, preferred_element_type=jnp.float32)


def _mega_body(maxl, dur_ref, x_ref, pt_ref, et_ref, pb_ref, eb_ref,
               emb2_ref, w1pe_ref, b1pe_ref, *refs):
    dur_p = [r[...] for r in refs[0:9]]
    pit_t = [r[...] for r in refs[9:16]]
    ene_t = [r[...] for r in refs[16:23]]
    out_ref, ld_ref, pp_ref, ep_ref, csum_ref = refs[23:28]

    s = dur_ref.shape[-1]
    dur = dur_ref[0].astype(jnp.float32)
    i = jax.lax.broadcasted_iota(jnp.int32, (s, s), 0)
    j = jax.lax.broadcasted_iota(jnp.int32, (s, s), 1)
    u = (i <= j).astype(jnp.float32)
    cs = jnp.dot(dur, u, preferred_element_type=jnp.float32)
    prev = cs - dur
    pos = jax.lax.broadcasted_iota(jnp.int32, (maxl, s), 0).astype(jnp.float32)
    e = ((prev <= pos) & (pos < cs)).astype(jnp.float32)
    x_exp = jnp.dot(e, x_ref[0], preferred_element_type=jnp.float32)
    csum_ref[0] = cs.astype(jnp.int32)

    ld_ref[0] = _vp_block(x_ref[0], *dur_p)

    f = pit_t[4].shape[1]
    xce = _shift_cat(x_exp)
    h12 = jnp.maximum(
        jnp.dot(xce, w1pe_ref[...], preferred_element_type=jnp.float32)
        + b1pe_ref[...], 0.0)

    def _vp_tail(h1, g1, be1, w2c, b2, g2, be2, lw):
        h = _ln(h1, g1, be1)
        h = jnp.dot(_shift_cat(h), w2c, preferred_element_type=jnp.float32) + b2
        h = _ln(jnp.maximum(h, 0.0), g2, be2)
        return jnp.dot(h, lw, preferred_element_type=jnp.float32)

    pp_ref[0] = _vp_tail(h12[:, :f], *pit_t)
    ep_ref[0] = _vp_tail(h12[:, f:], *ene_t)

    nb = pb_ref.shape[1]
    lane = jax.lax.broadcasted_iota(jnp.int32, (maxl, nb), 1).astype(jnp.float32)

    def onehot_of(t_ref, bins_ref):
        v = t_ref[0]
        hit = (bins_ref[...] < v).astype(jnp.float32)
        idx = jnp.sum(hit, axis=1, keepdims=True)
        return (lane == idx).astype(jnp.float32)

    oh2 = jnp.concatenate([onehot_of(pt_ref, pb_ref),
                           onehot_of(et_ref, eb_ref)], axis=1)
    out_ref[0] = x_exp + jnp.dot(oh2, emb2_ref[...],
                                 preferred_element_type=jnp.float32)


def _full(shape, dtype=jnp.float32):
    return pl.BlockSpec(shape, lambda b: (0,) * len(shape))


def _batched(shape):
    return pl.BlockSpec((1,) + shape, lambda b: (b,) + (0,) * len(shape))


def _vp_flat(p, h):
    f = p['w1'].shape[2]
    return (
        p['w1'].reshape(3 * h, f),
        p['b1'].reshape(1, f),
        p['g1'].reshape(1, f),
        p['be1'].reshape(1, f),
        p['w2'].reshape(3 * f, f),
        p['b2'].reshape(1, f),
        p['g2'].reshape(1, f),
        p['be2'].reshape(1, f),
        p['lw'],
    )


def kernel(x, src_mask, mel_mask, duration_target, pitch_target,
           energy_target, max_len, params):
    b, s, h = x.shape
    maxl = mel_mask.shape[1]
    nb = params['pitch_emb'].shape[0]

    cp = pltpu.CompilerParams(dimension_semantics=("parallel",))

    dur_p = _vp_flat(params['dur'], h)
    pit_p = _vp_flat(params['pitch'], h)
    ene_p = _vp_flat(params['energy'], h)
    w1pe = jnp.concatenate([pit_p[0], ene_p[0]], axis=1)
    b1pe = jnp.concatenate([pit_p[1], ene_p[1]], axis=1)
    pit_t = pit_p[2:]
    ene_t = ene_p[2:]
    emb2 = jnp.concatenate([params['pitch_emb'], params['energy_emb']],
                           axis=0)
    param_specs = [_full(a.shape) for a in dur_p + pit_t + ene_t]

    inf_pad = jnp.full((1,), jnp.inf, jnp.float32)
    pbins = jnp.concatenate([params['pitch_bins'], inf_pad]).reshape(1, nb)
    ebins = jnp.concatenate([params['energy_bins'], inf_pad]).reshape(1, nb)

    out, log_dur, pitch_pred, energy_pred, csum = pl.pallas_call(
        functools.partial(_mega_body, maxl),
        grid=(b,),
        in_specs=[_batched((1, s)), _batched((s, h)),
                  _batched((maxl, 1)), _batched((maxl, 1)),
                  _full((1, nb)), _full((1, nb)),
                  _full((2 * nb, h)), _full(w1pe.shape), _full(b1pe.shape)]
                 + param_specs,
        out_specs=[_batched((maxl, h)), _batched((s, 1)),
                   _batched((maxl, 1)), _batched((maxl, 1)),
                   _batched((1, s))],
        out_shape=[jax.ShapeDtypeStruct((b, maxl, h), jnp.float32),
                   jax.ShapeDtypeStruct((b, s, 1), jnp.float32),
                   jax.ShapeDtypeStruct((b, maxl, 1), jnp.float32),
                   jax.ShapeDtypeStruct((b, maxl, 1), jnp.float32),
                   jax.ShapeDtypeStruct((b, 1, s), jnp.int32)],
        compiler_params=cp,
    )(duration_target.reshape(b, 1, s), x,
      pitch_target.reshape(b, maxl, 1), energy_target.reshape(b, maxl, 1),
      pbins, ebins, emb2, w1pe, b1pe,
      *dur_p, *pit_t, *ene_t)

    mel_len = csum[:, 0, -1]
    log_dur = log_dur[:, :, 0] + params['dur']['lb'][0]
    pitch_pred = pitch_pred[:, :, 0] + params['pitch']['lb'][0]
    energy_pred = energy_pred[:, :, 0] + params['energy']['lb'][0]
    log_dur = jnp.where(src_mask, 0.0, log_dur)
    pitch_pred = jnp.where(mel_mask, 0.0, pitch_pred)
    energy_pred = jnp.where(mel_mask, 0.0, energy_pred)

    return (out, log_dur, pitch_pred, energy_pred, mel_len, mel_mask)


# speedup vs baseline: 1.2877x; 1.2877x over previous
import functools

import jax
import jax.numpy as jnp
from jax.experimental import pallas as pl
from jax.experimental.pallas import tpu as pltpu


def _ln(h, g, b):
    m = jnp.mean(h, axis=-1, keepdims=True)
    v = jnp.mean(h * h, axis=-1, keepdims=True) - m * m
    return (h - m) * jax.lax.rsqrt(v + 1e-5) * g + b


def _shift_cat(x):
    z = jnp.zeros((1, x.shape[1]), x.dtype)
    prv = jnp.concatenate([z, x[:-1]], axis=0)
    nxt = jnp.concatenate([x[1:], z], axis=0)
    return jnp.concatenate([prv, x, nxt], axis=1)


def _vp_block(x, w1c, b1, g1, be1, w2c, b2, g2, be2, lw):
    h = jnp.dot(_shift_cat(x), w1c, preferred_element_type=jnp.float32) + b1
    h = _ln(jnp.maximum(h, 0.0), g1, be1)
    h = jnp.dot(_shift_cat(h), w2c, preferred_element_type=jnp.float32) + b2
    h = _ln(jnp.maximum(h, 0.0), g2, be2)
    return jnp.dot(h, lw, preferred_element_type=jnp.float32)


def _mega_body(maxl, dur_ref, x_ref, pt_ref, et_ref, pb_ref, eb_ref,
               emb2_ref, w1pe_ref, b1pe_ref, *refs):
    dur_p = [r[...] for r in refs[0:9]]
    pit_t = [r[...] for r in refs[9:16]]
    ene_t = [r[...] for r in refs[16:23]]
    out_ref, ld_ref, pp_ref, ep_ref, csum_ref = refs[23:28]

    s = dur_ref.shape[-1]
    dur = dur_ref[0].astype(jnp.float32)
    i = jax.lax.broadcasted_iota(jnp.int32, (s, s), 0)
    j = jax.lax.broadcasted_iota(jnp.int32, (s, s), 1)
    u = (i <= j).astype(jnp.float32)
    cs = jnp.dot(dur, u, preferred_element_type=jnp.float32)
    prev = cs - dur
    pos = jax.lax.broadcasted_iota(jnp.int32, (maxl, s), 0).astype(jnp.float32)
    e = ((prev <= pos) & (pos < cs)).astype(jnp.float32)
    x_exp = jnp.dot(e, x_ref[0], preferred_element_type=jnp.float32)
    csum_ref[0] = cs.astype(jnp.int32)

    ld_ref[0] = _vp_block(x_ref[0], *dur_p)

    f = pit_t[4].shape[1]
    xce = _shift_cat(x_exp)
    h12 = jnp.maximum(
        jnp.dot(xce, w1pe_ref[...], preferred_element_type=jnp.float32)
        + b1pe_ref[...], 0.0)

    def _vp_tail(h1, g1, be1, w2c, b2, g2, be2, lw):
        h = _ln(h1, g1, be1)
        h = jnp.dot(_shift_cat(h), w2c, preferred_element_type=jnp.float32) + b2
        h = _ln(jnp.maximum(h, 0.0), g2, be2)
        return jnp.dot(h, lw, preferred_element_type=jnp.float32)

    pp_ref[0] = _vp_tail(h12[:, :f], *pit_t)
    ep_ref[0] = _vp_tail(h12[:, f:], *ene_t)

    nb = pb_ref.shape[1]
    lane = jax.lax.broadcasted_iota(jnp.int32, (maxl, nb), 1).astype(jnp.float32)

    def onehot_of(t_ref, bins_ref):
        v = t_ref[0]
        hit = (bins_ref[...] < v).astype(jnp.float32)
        idx = jnp.sum(hit, axis=1, keepdims=True)
        return (lane == idx).astype(jnp.float32)

    oh2 = jnp.concatenate([onehot_of(pt_ref, pb_ref),
                           onehot_of(et_ref, eb_ref)], axis=1)
    out_ref[0] = x_exp + jnp.dot(oh2, emb2_ref[...],
                                 preferred_element_type=jnp.float32)


def _full(shape, dtype=jnp.float32):
    return pl.BlockSpec(shape, lambda b: (0,) * len(shape))


def _batched(shape):
    return pl.BlockSpec((1,) + shape, lambda b: (b,) + (0,) * len(shape))


def _vp_flat(p, h):
    f = p['w1'].shape[2]
    return (
        p['w1'].reshape(3 * h, f),
        p['b1'].reshape(1, f),
        p['g1'].reshape(1, f),
        p['be1'].reshape(1, f),
        p['w2'].reshape(3 * f, f),
        p['b2'].reshape(1, f),
        p['g2'].reshape(1, f),
        p['be2'].reshape(1, f),
        p['lw'],
    )


def kernel(x, src_mask, mel_mask, duration_target, pitch_target,
           energy_target, max_len, params):
    b, s, h = x.shape
    maxl = mel_mask.shape[1]
    nb = params['pitch_emb'].shape[0]

    cp = pltpu.CompilerParams(dimension_semantics=("parallel",))

    dur_p = _vp_flat(params['dur'], h)
    pit_p = _vp_flat(params['pitch'], h)
    ene_p = _vp_flat(params['energy'], h)
    w1pe = jnp.concatenate([pit_p[0], ene_p[0]], axis=1)
    b1pe = jnp.concatenate([pit_p[1], ene_p[1]], axis=1)
    pit_t = pit_p[2:]
    ene_t = ene_p[2:]
    emb2 = jnp.concatenate([params['pitch_emb'], params['energy_emb']],
                           axis=0)
    param_specs = [_full(a.shape) for a in dur_p + pit_t + ene_t]

    inf_pad = jnp.full((1,), jnp.inf, jnp.float32)
    pbins = jnp.concatenate([params['pitch_bins'], inf_pad]).reshape(1, nb)
    ebins = jnp.concatenate([params['energy_bins'], inf_pad]).reshape(1, nb)

    out, log_dur, pitch_pred, energy_pred, csum = pl.pallas_call(
        functools.partial(_mega_body, maxl),
        grid=(b,),
        in_specs=[_batched((1, s)), _batched((s, h)),
                  _batched((maxl, 1)), _batched((maxl, 1)),
                  _full((1, nb)), _full((1, nb)),
                  _full((2 * nb, h)), _full(w1pe.shape), _full(b1pe.shape)]
                 + param_specs,
        out_specs=[_batched((maxl, h)), _batched((s, 1)),
                   _batched((maxl, 1)), _batched((maxl, 1)),
                   _batched((1, s))],
        out_shape=[jax.ShapeDtypeStruct((b, maxl, h), jnp.float32),
                   jax.ShapeDtypeStruct((b, s, 1), jnp.float32),
                   jax.ShapeDtypeStruct((b, maxl, 1), jnp.float32),
                   jax.ShapeDtypeStruct((b, maxl, 1), jnp.float32),
                   jax.ShapeDtypeStruct((b, 1, s), jnp.int32)],
        compiler_params=cp,
    )(duration_target.reshape(b, 1, s), x,
      pitch_target.reshape(b, maxl, 1), energy_target.reshape(b, maxl, 1),
      pbins, ebins, emb2, w1pe, b1pe,
      *dur_p, *pit_t, *ene_t)

    mel_len = csum[:, 0, -1]
    log_dur = log_dur[:, :, 0] + params['dur']['lb'][0]
    pitch_pred = pitch_pred[:, :, 0] + params['pitch']['lb'][0]
    energy_pred = energy_pred[:, :, 0] + params['energy']['lb'][0]
    log_dur = jnp.where(src_mask, 0.0, log_dur)
    pitch_pred = jnp.where(mel_mask, 0.0, pitch_pred)
    energy_pred = jnp.where(mel_mask, 0.0, energy_pred)

    return (out, log_dur, pitch_pred, energy_pred, mel_len, mel_mask)
